# baseline (speedup 1.0000x reference)
import functools

import jax
import jax.numpy as jnp
from jax import lax
from jax.experimental import pallas as pl
from jax.experimental.pallas import tpu as pltpu
from jax.experimental.pallas import tpu_sc as plsc

N = 10000
E = 320000
F = 128
H = 128
C = 10
G = 128

NC = 2
NS = 16
CHUNK = 128
CW = 80
CPP = 40
PP = CPP
NROWS = NC * NS * CW
E_PAD = NROWS * CHUNK
N_P = 10240
RPT = N_P // NS

_mesh = plsc.VectorSubcoreMesh(
    core_axis_name="c", subcore_axis_name="s", num_cores=NC, num_subcores=NS
)


@functools.partial(
    pl.kernel,
    out_type=jax.ShapeDtypeStruct((NC, N_P, H), jnp.float32),
    mesh=_mesh,
    scratch_types=[
        pltpu.VMEM((PP, CHUNK), jnp.int32),
        pltpu.VMEM((CHUNK, H), jnp.float32),
        pltpu.VMEM_SHARED((N_P, H), jnp.float32),
    ],
)
def _sc_degree(dstb, ones_hbm, zeros, out, didx, ones_v, acc):
    c = lax.axis_index("c")
    s = lax.axis_index("s")
    pltpu.sync_copy(zeros.at[pl.ds(s * RPT, RPT)], acc.at[pl.ds(s * RPT, RPT)])
    pltpu.sync_copy(ones_hbm, ones_v)
    wid = s * NC + c
    plsc.subcore_barrier()

    for p in range(CW // CPP):
        row0 = pl.multiple_of(wid * CW + p * CPP, 8)
        pltpu.sync_copy(dstb.at[pl.ds(row0, CPP)], didx)

        def body(l, carry):
            pltpu.sync_copy(ones_v, acc.at[didx.at[l]], add=True)
            return carry

        lax.fori_loop(0, CPP, body, 0)

    plsc.subcore_barrier()
    pltpu.sync_copy(acc.at[pl.ds(s * RPT, RPT)], out.at[c, pl.ds(s * RPT, RPT)])


@functools.partial(
    pl.kernel,
    out_type=jax.ShapeDtypeStruct((NC, N_P, H), jnp.float32),
    mesh=_mesh,
    scratch_types=[
        pltpu.VMEM((PP, CHUNK), jnp.int32),
        pltpu.VMEM((PP, CHUNK), jnp.int32),
        pltpu.VMEM((2, CHUNK, H), jnp.float32),
        pltpu.VMEM_SHARED((N_P, H), jnp.float32),
        pltpu.SemaphoreType.DMA,
        pltpu.SemaphoreType.DMA,
    ],
)
def _sc_agg(y, srcb, dstb, zeros, out, sidx, didx, rows, acc, sem0, sem1):
    c = lax.axis_index("c")
    s = lax.axis_index("s")
    pltpu.sync_copy(zeros.at[pl.ds(s * RPT, RPT)], acc.at[pl.ds(s * RPT, RPT)])
    wid = s * NC + c
    plsc.subcore_barrier()

    for p in range(CW // CPP):
        row0 = pl.multiple_of(wid * CW + p * CPP, 8)
        pltpu.sync_copy(srcb.at[pl.ds(row0, CPP)], sidx)
        pltpu.sync_copy(dstb.at[pl.ds(row0, CPP)], didx)
        pltpu.async_copy(y.at[sidx.at[0]], rows.at[0], sem0)
        pltpu.async_copy(y.at[sidx.at[1]], rows.at[1], sem1)

        def body2(l2, carry):
            def step(l, slot, sem):
                pltpu.make_async_copy(
                    y.at[sidx.at[l]], rows.at[slot], sem
                ).wait()
                pltpu.sync_copy(rows.at[slot], acc.at[didx.at[l]], add=True)

                @pl.when(l + 2 < CPP)
                def _():
                    pltpu.async_copy(
                        y.at[sidx.at[l + 2]], rows.at[slot], sem
                    )

            step(l2 * 2, 0, sem0)
            step(l2 * 2 + 1, 1, sem1)
            return carry

        lax.fori_loop(0, CPP // 2, body2, 0)

    plsc.subcore_barrier()
    pltpu.sync_copy(acc.at[pl.ds(s * RPT, RPT)], out.at[c, pl.ds(s * RPT, RPT)])


BN = 1024
NBLK = N_P // BN


def _tc_pre_body(x_ref, w_ref, dinv_ref, y_ref):
    y_ref[...] = dinv_ref[...] * jnp.dot(
        x_ref[...], w_ref[...], preferred_element_type=jnp.float32
    )


def _tc_pre(x, W, dinvb):
    return pl.pallas_call(
        _tc_pre_body,
        grid=(NBLK,),
        in_specs=[
            pl.BlockSpec((BN, F), lambda i: (i, 0)),
            pl.BlockSpec((F, H), lambda i: (0, 0)),
            pl.BlockSpec((BN, H), lambda i: (i, 0)),
        ],
        out_specs=pl.BlockSpec((BN, H), lambda i: (i, 0)),
        out_shape=jax.ShapeDtypeStruct((N_P, H), jnp.float32),
    )(x, W, dinvb)


def _tc_mid_body(a_ref, y_ref, dinv_ref, b_ref, w_ref, o_ref):
    h = dinv_ref[...] * (a_ref[0] + a_ref[1] + y_ref[...]) + b_ref[...]
    h = jnp.maximum(h, 0.0)
    o_ref[...] = dinv_ref[...] * jnp.dot(
        h, w_ref[...], preferred_element_type=jnp.float32
    )


def _tc_mid(A, y, dinvb, b, Wn):
    return pl.pallas_call(
        _tc_mid_body,
        grid=(NBLK,),
        in_specs=[
            pl.BlockSpec((NC, BN, H), lambda i: (0, i, 0)),
            pl.BlockSpec((BN, H), lambda i: (i, 0)),
            pl.BlockSpec((BN, H), lambda i: (i, 0)),
            pl.BlockSpec((1, H), lambda i: (0, 0)),
            pl.BlockSpec((H, H), lambda i: (0, 0)),
        ],
        out_specs=pl.BlockSpec((BN, H), lambda i: (i, 0)),
        out_shape=jax.ShapeDtypeStruct((N_P, H), jnp.float32),
    )(A, y, dinvb, b, Wn)


def _tc_final_body(
    a_ref, y_ref, dinv_ref, b_ref, batch_ref, wl_ref, bl_ref, o_ref, sums, cnts
):
    i = pl.program_id(0)

    @pl.when(i == 0)
    def _():
        sums[...] = jnp.zeros_like(sums)
        cnts[...] = jnp.zeros_like(cnts)

    h = dinv_ref[...] * (a_ref[0] + a_ref[1] + y_ref[...]) + b_ref[...]
    h = jnp.maximum(h, 0.0)
    gi = lax.broadcasted_iota(jnp.int32, (G, BN), 0)
    oh = jnp.where(gi == batch_ref[...], 1.0, 0.0)
    sums[...] += jnp.dot(oh, h, preferred_element_type=jnp.float32)
    cnts[...] += jnp.dot(
        oh, jnp.ones((BN, H), jnp.float32), preferred_element_type=jnp.float32
    )

    @pl.when(i == NBLK - 1)
    def _():
        pooled = sums[...] / jnp.maximum(cnts[...], 1.0)
        o_ref[...] = (
            jnp.dot(pooled, wl_ref[...], preferred_element_type=jnp.float32)
            + bl_ref[...]
        )


def _tc_final(A, y, dinvb, b, batch_p, Wl, bl):
    return pl.pallas_call(
        _tc_final_body,
        grid=(NBLK,),
        in_specs=[
            pl.BlockSpec((NC, BN, H), lambda i: (0, i, 0)),
            pl.BlockSpec((BN, H), lambda i: (i, 0)),
            pl.BlockSpec((BN, H), lambda i: (i, 0)),
            pl.BlockSpec((1, H), lambda i: (0, 0)),
            pl.BlockSpec((1, BN), lambda i: (0, i)),
            pl.BlockSpec((H, C), lambda i: (0, 0)),
            pl.BlockSpec((1, C), lambda i: (0, 0)),
        ],
        out_specs=pl.BlockSpec((G, C), lambda i: (0, 0)),
        out_shape=jax.ShapeDtypeStruct((G, C), jnp.float32),
        scratch_shapes=[
            pltpu.VMEM((G, H), jnp.float32),
            pltpu.VMEM((G, H), jnp.float32),
        ],
    )(A, y, dinvb, b, batch_p, Wl, bl)


def kernel(x, edge_index, batch, W1, b1, W2, b2, W3, b3, Wl, bl):
    f32 = jnp.float32
    src = edge_index[0]
    dst = edge_index[1]
    pad = E_PAD - E
    srcb = jnp.concatenate([src, jnp.full((pad,), N, jnp.int32)]).reshape(
        NROWS, CHUNK
    )
    dstb = jnp.concatenate([dst, jnp.full((pad,), N, jnp.int32)]).reshape(
        NROWS, CHUNK
    )
    zeros_nh = jnp.zeros((N_P, H), f32)
    ones_ch = jnp.ones((CHUNK, H), f32)

    cnt = _sc_degree(dstb, ones_ch, zeros_nh)
    deg = cnt[0, :N, 0] + cnt[1, :N, 0] + 1.0
    dinv = lax.rsqrt(deg)
    dinvb = jnp.concatenate(
        [jnp.broadcast_to(dinv[:, None], (N, H)), jnp.zeros((N_P - N, H), f32)]
    )

    x_p = jnp.concatenate([x, jnp.zeros((N_P - N, F), f32)])
    batch_p = jnp.concatenate(
        [batch, jnp.full((N_P - N,), G, jnp.int32)]
    ).reshape(1, N_P)

    y1 = _tc_pre(x_p, W1, dinvb)
    A1 = _sc_agg(y1, srcb, dstb, zeros_nh)
    y2 = _tc_mid(A1, y1, dinvb, b1.reshape(1, H), W2)
    A2 = _sc_agg(y2, srcb, dstb, zeros_nh)
    y3 = _tc_mid(A2, y2, dinvb, b2.reshape(1, H), W3)
    A3 = _sc_agg(y3, srcb, dstb, zeros_nh)
    return _tc_final(A3, y3, dinvb, b3.reshape(1, H), batch_p, Wl, bl.reshape(1, C))


# speedup vs baseline: 1.5560x; 1.5560x over previous
import functools

import jax
import jax.numpy as jnp
from jax import lax
from jax.experimental import pallas as pl
from jax.experimental.pallas import tpu as pltpu
from jax.experimental.pallas import tpu_sc as plsc

N = 10000
E = 320000
F = 128
H = 128
C = 10
G = 128

NC = 2
NS = 16
CHUNK = 128
NW = NC * NS
CW = -(-E // (NW * CHUNK))
CPP = 40
E_PAD = NW * CW * CHUNK
N_P = 10240
RPT = N_P // NS

_mesh = plsc.VectorSubcoreMesh(
    core_axis_name="c", subcore_axis_name="s", num_cores=NC, num_subcores=NS
)


@functools.partial(
    pl.kernel,
    out_type=jax.ShapeDtypeStruct((NC, N_P, H), jnp.float32),
    mesh=_mesh,
    scratch_types=[
        pltpu.VMEM((CW, CHUNK), jnp.int32),
        pltpu.VMEM((CHUNK, H), jnp.float32),
        pltpu.VMEM_SHARED((N_P, H), jnp.float32),
    ],
)
def _sc_degree(dstb, ones_hbm, zeros, out, didx, ones_v, acc):
    c = lax.axis_index("c")
    s = lax.axis_index("s")
    wid = s * NC + c
    pltpu.sync_copy(zeros.at[pl.ds(s * RPT, RPT)], acc.at[pl.ds(s * RPT, RPT)])
    pltpu.sync_copy(ones_hbm, ones_v)
    pltpu.sync_copy(dstb.at[wid], didx)
    plsc.subcore_barrier()

    def body(j, carry):
        pltpu.sync_copy(ones_v, acc.at[didx.at[j]], add=True)
        return carry

    lax.fori_loop(0, CW, body, 0)
    plsc.subcore_barrier()
    pltpu.sync_copy(acc.at[pl.ds(s * RPT, RPT)], out.at[c, pl.ds(s * RPT, RPT)])


@functools.partial(
    pl.kernel,
    out_type=jax.ShapeDtypeStruct((NC, N_P, H), jnp.float32),
    mesh=_mesh,
    scratch_types=[
        pltpu.VMEM((CPP, CHUNK), jnp.int32),
        pltpu.VMEM((CPP, CHUNK), jnp.int32),
        pltpu.VMEM((2, CHUNK, H), jnp.float32),
        pltpu.VMEM_SHARED((N_P, H), jnp.float32),
        pltpu.SemaphoreType.DMA,
    ],
)
def _sc_agg(y, srcb, dstb, zeros, out, sidx, didx, rows, acc, gsem):
    c = lax.axis_index("c")
    s = lax.axis_index("s")
    wid = s * NC + c
    pltpu.sync_copy(zeros.at[pl.ds(s * RPT, RPT)], acc.at[pl.ds(s * RPT, RPT)])
    plsc.subcore_barrier()

    for p, np_ in ((0, CPP), (1, CW - CPP)):
        pltpu.sync_copy(
            srcb.at[wid].at[pl.ds(p * CPP, np_)], sidx.at[pl.ds(0, np_)]
        )
        pltpu.sync_copy(
            dstb.at[wid].at[pl.ds(p * CPP, np_)], didx.at[pl.ds(0, np_)]
        )
        pltpu.async_copy(y.at[sidx.at[0]], rows.at[0], gsem)

        def body(l, carry):
            slot = lax.rem(l, 2)
            nslot = lax.rem(l + 1, 2)
            pltpu.make_async_copy(y.at[sidx.at[l]], rows.at[slot], gsem).wait()

            @pl.when(l + 1 < np_)
            def _():
                pltpu.async_copy(y.at[sidx.at[l + 1]], rows.at[nslot], gsem)

            pltpu.sync_copy(rows.at[slot], acc.at[didx.at[l]], add=True)
            return carry

        lax.fori_loop(0, np_, body, 0)

    plsc.subcore_barrier()
    pltpu.sync_copy(acc.at[pl.ds(s * RPT, RPT)], out.at[c, pl.ds(s * RPT, RPT)])


BN = 1024
NBLK = N_P // BN


def _tc_pre_body(x_ref, w_ref, dinv_ref, y_ref):
    y_ref[...] = dinv_ref[...] * jnp.dot(
        x_ref[...], w_ref[...], preferred_element_type=jnp.float32
    )


def _tc_pre(x, W, dinvb):
    return pl.pallas_call(
        _tc_pre_body,
        grid=(NBLK,),
        in_specs=[
            pl.BlockSpec((BN, F), lambda i: (i, 0)),
            pl.BlockSpec((F, H), lambda i: (0, 0)),
            pl.BlockSpec((BN, H), lambda i: (i, 0)),
        ],
        out_specs=pl.BlockSpec((BN, H), lambda i: (i, 0)),
        out_shape=jax.ShapeDtypeStruct((N_P, H), jnp.float32),
    )(x, W, dinvb)


def _tc_mid_body(a_ref, y_ref, dinv_ref, b_ref, w_ref, o_ref):
    h = dinv_ref[...] * (a_ref[0] + a_ref[1] + y_ref[...]) + b_ref[...]
    h = jnp.maximum(h, 0.0)
    o_ref[...] = dinv_ref[...] * jnp.dot(
        h, w_ref[...], preferred_element_type=jnp.float32
    )


def _tc_mid(A, y, dinvb, b, Wn):
    return pl.pallas_call(
        _tc_mid_body,
        grid=(NBLK,),
        in_specs=[
            pl.BlockSpec((NC, BN, H), lambda i: (0, i, 0)),
            pl.BlockSpec((BN, H), lambda i: (i, 0)),
            pl.BlockSpec((BN, H), lambda i: (i, 0)),
            pl.BlockSpec((1, H), lambda i: (0, 0)),
            pl.BlockSpec((H, H), lambda i: (0, 0)),
        ],
        out_specs=pl.BlockSpec((BN, H), lambda i: (i, 0)),
        out_shape=jax.ShapeDtypeStruct((N_P, H), jnp.float32),
    )(A, y, dinvb, b, Wn)


def _tc_final_body(
    a_ref, y_ref, dinv_ref, b_ref, batch_ref, wl_ref, bl_ref, o_ref, sums, cnts
):
    i = pl.program_id(0)

    @pl.when(i == 0)
    def _():
        sums[...] = jnp.zeros_like(sums)
        cnts[...] = jnp.zeros_like(cnts)

    h = dinv_ref[...] * (a_ref[0] + a_ref[1] + y_ref[...]) + b_ref[...]
    h = jnp.maximum(h, 0.0)
    gi = lax.broadcasted_iota(jnp.int32, (G, BN), 0)
    oh = jnp.where(gi == batch_ref[...], 1.0, 0.0)
    sums[...] += jnp.dot(oh, h, preferred_element_type=jnp.float32)
    cnts[...] += jnp.dot(
        oh, jnp.ones((BN, H), jnp.float32), preferred_element_type=jnp.float32
    )

    @pl.when(i == NBLK - 1)
    def _():
        pooled = sums[...] / jnp.maximum(cnts[...], 1.0)
        o_ref[...] = (
            jnp.dot(pooled, wl_ref[...], preferred_element_type=jnp.float32)
            + bl_ref[...]
        )


def _tc_final(A, y, dinvb, b, batch_p, Wl, bl):
    return pl.pallas_call(
        _tc_final_body,
        grid=(NBLK,),
        in_specs=[
            pl.BlockSpec((NC, BN, H), lambda i: (0, i, 0)),
            pl.BlockSpec((BN, H), lambda i: (i, 0)),
            pl.BlockSpec((BN, H), lambda i: (i, 0)),
            pl.BlockSpec((1, H), lambda i: (0, 0)),
            pl.BlockSpec((1, BN), lambda i: (0, i)),
            pl.BlockSpec((H, C), lambda i: (0, 0)),
            pl.BlockSpec((1, C), lambda i: (0, 0)),
        ],
        out_specs=pl.BlockSpec((G, C), lambda i: (0, 0)),
        out_shape=jax.ShapeDtypeStruct((G, C), jnp.float32),
        scratch_shapes=[
            pltpu.VMEM((G, H), jnp.float32),
            pltpu.VMEM((G, H), jnp.float32),
        ],
    )(A, y, dinvb, b, batch_p, Wl, bl)


def kernel(x, edge_index, batch, W1, b1, W2, b2, W3, b3, Wl, bl):
    f32 = jnp.float32
    src = edge_index[0]
    dst = edge_index[1]
    pad = E_PAD - E
    srcb = jnp.concatenate([src, jnp.full((pad,), N, jnp.int32)]).reshape(
        NW, CW, CHUNK
    )
    dstb = jnp.concatenate([dst, jnp.full((pad,), N, jnp.int32)]).reshape(
        NW, CW, CHUNK
    )
    zeros_nh = jnp.zeros((N_P, H), f32)
    ones_ch = jnp.ones((CHUNK, H), f32)

    cnt = _sc_degree(dstb, ones_ch, zeros_nh)
    deg = cnt[0, :N, 0] + cnt[1, :N, 0] + 1.0
    dinv = lax.rsqrt(deg)
    dinvb = jnp.concatenate(
        [jnp.broadcast_to(dinv[:, None], (N, H)), jnp.zeros((N_P - N, H), f32)]
    )

    x_p = jnp.concatenate([x, jnp.zeros((N_P - N, F), f32)])
    batch_p = jnp.concatenate(
        [batch, jnp.full((N_P - N,), G, jnp.int32)]
    ).reshape(1, N_P)

    y1 = _tc_pre(x_p, W1, dinvb)
    A1 = _sc_agg(y1, srcb, dstb, zeros_nh)
    y2 = _tc_mid(A1, y1, dinvb, b1.reshape(1, H), W2)
    A2 = _sc_agg(y2, srcb, dstb, zeros_nh)
    y3 = _tc_mid(A2, y2, dinvb, b2.reshape(1, H), W3)
    A3 = _sc_agg(y3, srcb, dstb, zeros_nh)
    return _tc_final(A3, y3, dinvb, b3.reshape(1, H), batch_p, Wl, bl.reshape(1, C))


# speedup vs baseline: 1.6605x; 1.0672x over previous
import functools

import jax
import jax.numpy as jnp
from jax import lax
from jax.experimental import pallas as pl
from jax.experimental.pallas import tpu as pltpu
from jax.experimental.pallas import tpu_sc as plsc

N = 10000
E = 320000
F = 128
H = 128
C = 10
G = 128

NC = 2
NS = 16
CHUNK = 128
NW = NC * NS
CW = -(-E // (NW * CHUNK))
CPP = 40
E_PAD = NW * CW * CHUNK
N_P = 10240
RPT = N_P // NS

_mesh = plsc.VectorSubcoreMesh(
    core_axis_name="c", subcore_axis_name="s", num_cores=NC, num_subcores=NS
)


@functools.partial(
    pl.kernel,
    out_type=jax.ShapeDtypeStruct((NC, N_P, H), jnp.float32),
    mesh=_mesh,
    scratch_types=[
        pltpu.VMEM((CW, CHUNK), jnp.int32),
        pltpu.VMEM((CHUNK, H), jnp.float32),
        pltpu.VMEM_SHARED((N_P, H), jnp.float32),
    ],
)
def _sc_degree(dstb, ones_hbm, zeros, out, didx, ones_v, acc):
    c = lax.axis_index("c")
    s = lax.axis_index("s")
    wid = s * NC + c
    pltpu.sync_copy(zeros.at[pl.ds(s * RPT, RPT)], acc.at[pl.ds(s * RPT, RPT)])
    pltpu.sync_copy(ones_hbm, ones_v)
    pltpu.sync_copy(dstb.at[wid], didx)
    plsc.subcore_barrier()

    def body(j, carry):
        pltpu.sync_copy(ones_v, acc.at[didx.at[j]], add=True)
        return carry

    lax.fori_loop(0, CW, body, 0)
    plsc.subcore_barrier()
    pltpu.sync_copy(acc.at[pl.ds(s * RPT, RPT)], out.at[c, pl.ds(s * RPT, RPT)])


@functools.partial(
    pl.kernel,
    out_type=jax.ShapeDtypeStruct((NC, N_P, H), jnp.float32),
    mesh=_mesh,
    scratch_types=[
        pltpu.VMEM((CPP, CHUNK), jnp.int32),
        pltpu.VMEM((CPP, CHUNK), jnp.int32),
        pltpu.VMEM((2, CHUNK, H), jnp.float32),
        pltpu.VMEM_SHARED((N_P, H), jnp.float32),
        pltpu.SemaphoreType.DMA,
        pltpu.SemaphoreType.DMA,
    ],
)
def _sc_agg(y, srcb, dstb, zeros, out, sidx, didx, rows, acc, sem0, sem1):
    c = lax.axis_index("c")
    s = lax.axis_index("s")
    wid = s * NC + c
    pltpu.sync_copy(zeros.at[pl.ds(s * RPT, RPT)], acc.at[pl.ds(s * RPT, RPT)])
    plsc.subcore_barrier()

    for p, np_ in ((0, CPP), (1, CW - CPP)):
        pltpu.sync_copy(
            srcb.at[wid].at[pl.ds(p * CPP, np_)], sidx.at[pl.ds(0, np_)]
        )
        pltpu.sync_copy(
            dstb.at[wid].at[pl.ds(p * CPP, np_)], didx.at[pl.ds(0, np_)]
        )
        pltpu.async_copy(y.at[sidx.at[0]], rows.at[0], sem0)
        pltpu.async_copy(y.at[sidx.at[1]], rows.at[1], sem1)

        def step(l, slot, sem):
            pltpu.make_async_copy(y.at[sidx.at[l]], rows.at[slot], sem).wait()
            pltpu.sync_copy(rows.at[slot], acc.at[didx.at[l]], add=True)

            @pl.when(l + 2 < np_)
            def _():
                pltpu.async_copy(y.at[sidx.at[l + 2]], rows.at[slot], sem)

        def body2(l2, carry):
            step(l2 * 2, 0, sem0)
            step(l2 * 2 + 1, 1, sem1)
            return carry

        lax.fori_loop(0, np_ // 2, body2, 0)
        if np_ % 2:
            l = np_ - 1
            pltpu.make_async_copy(
                y.at[sidx.at[l]], rows.at[l % 2], (sem0, sem1)[l % 2]
            ).wait()
            pltpu.sync_copy(rows.at[l % 2], acc.at[didx.at[l]], add=True)

    plsc.subcore_barrier()
    pltpu.sync_copy(acc.at[pl.ds(s * RPT, RPT)], out.at[c, pl.ds(s * RPT, RPT)])


BN = 1024
NBLK = N_P // BN


def _tc_pre_body(x_ref, w_ref, dinv_ref, y_ref):
    y_ref[...] = dinv_ref[...] * jnp.dot(
        x_ref[...], w_ref[...], preferred_element_type=jnp.float32
    )


def _tc_pre(x, W, dinvb):
    return pl.pallas_call(
        _tc_pre_body,
        grid=(NBLK,),
        in_specs=[
            pl.BlockSpec((BN, F), lambda i: (i, 0)),
            pl.BlockSpec((F, H), lambda i: (0, 0)),
            pl.BlockSpec((BN, H), lambda i: (i, 0)),
        ],
        out_specs=pl.BlockSpec((BN, H), lambda i: (i, 0)),
        out_shape=jax.ShapeDtypeStruct((N_P, H), jnp.float32),
    )(x, W, dinvb)


def _tc_mid_body(a_ref, y_ref, dinv_ref, b_ref, w_ref, o_ref):
    h = dinv_ref[...] * (a_ref[0] + a_ref[1] + y_ref[...]) + b_ref[...]
    h = jnp.maximum(h, 0.0)
    o_ref[...] = dinv_ref[...] * jnp.dot(
        h, w_ref[...], preferred_element_type=jnp.float32
    )


def _tc_mid(A, y, dinvb, b, Wn):
    return pl.pallas_call(
        _tc_mid_body,
        grid=(NBLK,),
        in_specs=[
            pl.BlockSpec((NC, BN, H), lambda i: (0, i, 0)),
            pl.BlockSpec((BN, H), lambda i: (i, 0)),
            pl.BlockSpec((BN, H), lambda i: (i, 0)),
            pl.BlockSpec((1, H), lambda i: (0, 0)),
            pl.BlockSpec((H, H), lambda i: (0, 0)),
        ],
        out_specs=pl.BlockSpec((BN, H), lambda i: (i, 0)),
        out_shape=jax.ShapeDtypeStruct((N_P, H), jnp.float32),
    )(A, y, dinvb, b, Wn)


def _tc_final_body(
    a_ref, y_ref, dinv_ref, b_ref, batch_ref, wl_ref, bl_ref, o_ref, sums, cnts
):
    i = pl.program_id(0)

    @pl.when(i == 0)
    def _():
        sums[...] = jnp.zeros_like(sums)
        cnts[...] = jnp.zeros_like(cnts)

    h = dinv_ref[...] * (a_ref[0] + a_ref[1] + y_ref[...]) + b_ref[...]
    h = jnp.maximum(h, 0.0)
    gi = lax.broadcasted_iota(jnp.int32, (G, BN), 0)
    oh = jnp.where(gi == batch_ref[...], 1.0, 0.0)
    sums[...] += jnp.dot(oh, h, preferred_element_type=jnp.float32)
    cnts[...] += jnp.dot(
        oh, jnp.ones((BN, H), jnp.float32), preferred_element_type=jnp.float32
    )

    @pl.when(i == NBLK - 1)
    def _():
        pooled = sums[...] / jnp.maximum(cnts[...], 1.0)
        o_ref[...] = (
            jnp.dot(pooled, wl_ref[...], preferred_element_type=jnp.float32)
            + bl_ref[...]
        )


def _tc_final(A, y, dinvb, b, batch_p, Wl, bl):
    return pl.pallas_call(
        _tc_final_body,
        grid=(NBLK,),
        in_specs=[
            pl.BlockSpec((NC, BN, H), lambda i: (0, i, 0)),
            pl.BlockSpec((BN, H), lambda i: (i, 0)),
            pl.BlockSpec((BN, H), lambda i: (i, 0)),
            pl.BlockSpec((1, H), lambda i: (0, 0)),
            pl.BlockSpec((1, BN), lambda i: (0, i)),
            pl.BlockSpec((H, C), lambda i: (0, 0)),
            pl.BlockSpec((1, C), lambda i: (0, 0)),
        ],
        out_specs=pl.BlockSpec((G, C), lambda i: (0, 0)),
        out_shape=jax.ShapeDtypeStruct((G, C), jnp.float32),
        scratch_shapes=[
            pltpu.VMEM((G, H), jnp.float32),
            pltpu.VMEM((G, H), jnp.float32),
        ],
    )(A, y, dinvb, b, batch_p, Wl, bl)


def kernel(x, edge_index, batch, W1, b1, W2, b2, W3, b3, Wl, bl):
    f32 = jnp.float32
    src = edge_index[0]
    dst = edge_index[1]
    pad = E_PAD - E
    srcb = jnp.concatenate([src, jnp.full((pad,), N, jnp.int32)]).reshape(
        NW, CW, CHUNK
    )
    dstb = jnp.concatenate([dst, jnp.full((pad,), N, jnp.int32)]).reshape(
        NW, CW, CHUNK
    )
    zeros_nh = jnp.zeros((N_P, H), f32)
    ones_ch = jnp.ones((CHUNK, H), f32)

    cnt = _sc_degree(dstb, ones_ch, zeros_nh)
    deg = cnt[0, :N, 0] + cnt[1, :N, 0] + 1.0
    dinv = lax.rsqrt(deg)
    dinvb = jnp.concatenate(
        [jnp.broadcast_to(dinv[:, None], (N, H)), jnp.zeros((N_P - N, H), f32)]
    )

    x_p = jnp.concatenate([x, jnp.zeros((N_P - N, F), f32)])
    batch_p = jnp.concatenate(
        [batch, jnp.full((N_P - N,), G, jnp.int32)]
    ).reshape(1, N_P)

    y1 = _tc_pre(x_p, W1, dinvb)
    A1 = _sc_agg(y1, srcb, dstb, zeros_nh)
    y2 = _tc_mid(A1, y1, dinvb, b1.reshape(1, H), W2)
    A2 = _sc_agg(y2, srcb, dstb, zeros_nh)
    y3 = _tc_mid(A2, y2, dinvb, b2.reshape(1, H), W3)
    A3 = _sc_agg(y3, srcb, dstb, zeros_nh)
    return _tc_final(A3, y3, dinvb, b3.reshape(1, H), batch_p, Wl, bl.reshape(1, C))
